# baseline (speedup 1.0000x reference)
import functools

import jax
import jax.numpy as jnp
from jax import lax
from jax.experimental import pallas as pl
from jax.experimental.pallas import tpu as pltpu
from jax.experimental.pallas import tpu_sc as plsc

BATCH = 1024
ORIG_DIM = 16384
PROJ_DIM = 1024
C = 4
NC, NS = 2, 16
NT = NC * NS
ROWS_TILE = ORIG_DIM // NT
CH = 32
NCH = ROWS_TILE // CH
W = CH * PROJ_DIM
VPK = CH // 16

_mesh = plsc.VectorSubcoreMesh(core_axis_name="c", subcore_axis_name="s")


@functools.partial(
    pl.kernel,
    out_type=jax.ShapeDtypeStruct((ORIG_DIM * PROJ_DIM,), jnp.float32),
    mesh=_mesh,
    compiler_params=pltpu.CompilerParams(needs_layout_passes=False),
    scratch_types=[
        pltpu.VMEM((W,), jnp.float32),
        pltpu.VMEM((C * ROWS_TILE,), jnp.int32),
        pltpu.VMEM((C * ROWS_TILE,), jnp.float32),
    ],
)
def _build_s(idx_hbm, sgn_hbm, s_hbm, buf, idxv, sgnv):
    wid = lax.axis_index("s") * NC + lax.axis_index("c")
    j0 = wid * ROWS_TILE
    for k in range(C):
        pltpu.sync_copy(idx_hbm.at[pl.ds(k * ORIG_DIM + j0, ROWS_TILE)],
                        idxv.at[pl.ds(k * ROWS_TILE, ROWS_TILE)])
        pltpu.sync_copy(sgn_hbm.at[pl.ds(k * ORIG_DIM + j0, ROWS_TILE)],
                        sgnv.at[pl.ds(k * ROWS_TILE, ROWS_TILE)])

    zero16 = jnp.zeros((16,), jnp.float32)

    def _zero(i, _):
        buf[pl.ds(i * 16, 16)] = zero16
        return _

    lax.fori_loop(0, W // 16, _zero, 0)

    iota = lax.broadcasted_iota(jnp.int32, (16,), 0)
    scale = jnp.float32(0.5)

    def _chunk(c, _):
        tgts = []
        sgs = []
        for k in range(C):
            for v in range(VPK):
                off = k * ROWS_TILE + c * CH + v * 16
                idxs = idxv[pl.ds(off, 16)]
                sg = sgnv[pl.ds(off, 16)] * scale
                tgt = ((v * 16 + iota) << 10) + idxs
                plsc.addupdate_scatter(buf, [tgt], sg)
                tgts.append(tgt)
                sgs.append(sg)
        pltpu.sync_copy(buf, s_hbm.at[pl.ds((j0 + c * CH) * PROJ_DIM, W)])
        for tgt, sg in zip(tgts, sgs):
            plsc.addupdate_scatter(buf, [tgt], -sg)
        return _

    lax.fori_loop(0, NCH, _chunk, 0)


KB = 2048
N_STEPS = ORIG_DIM // KB


def _mm_body(x_ref, s_ref, o_ref, acc_ref):
    i = pl.program_id(0)

    @pl.when(i == 0)
    def _init():
        acc_ref[...] = jnp.zeros_like(acc_ref)

    xb = x_ref[...].astype(jnp.bfloat16)
    sb = s_ref[...].astype(jnp.bfloat16)
    acc_ref[...] += jnp.dot(xb, sb, preferred_element_type=jnp.float32)

    @pl.when(i == N_STEPS - 1)
    def _done():
        o_ref[...] = acc_ref[...]


def _matmul(x, s):
    return pl.pallas_call(
        _mm_body,
        grid=(N_STEPS,),
        in_specs=[
            pl.BlockSpec((BATCH, KB), lambda i: (0, i)),
            pl.BlockSpec((KB, PROJ_DIM), lambda i: (i, 0)),
        ],
        out_specs=pl.BlockSpec((BATCH, PROJ_DIM), lambda i: (0, 0)),
        out_shape=jax.ShapeDtypeStruct((BATCH, PROJ_DIM), jnp.float32),
        scratch_shapes=[pltpu.VMEM((BATCH, PROJ_DIM), jnp.float32)],
    )(x, s)


@jax.jit
def kernel(x, rand_indices, rand_signs):
    idx_t = rand_indices.T.reshape(-1)
    sgn_t = rand_signs.T.reshape(-1)
    s = _build_s(idx_t, sgn_t)
    return _matmul(x, s.reshape(ORIG_DIM, PROJ_DIM))


# speedup vs baseline: 1.9157x; 1.9157x over previous
import jax
import jax.numpy as jnp
from jax.experimental import pallas as pl
from jax.experimental.pallas import tpu as pltpu

BATCH = 1024
ORIG_DIM = 16384
PROJ_DIM = 1024
C = 4
KB = 2048
N_STEPS = ORIG_DIM // KB


def _body(x_ref, idx_ref, sgn_ref, o_ref, acc_ref, s_ref):
    i = pl.program_id(0)

    @pl.when(i == 0)
    def _init():
        acc_ref[...] = jnp.zeros_like(acc_ref)

    @pl.when(i < N_STEPS)
    def _build():
        idx = idx_ref[...]
        sgn = sgn_ref[...] * (1.0 / jnp.sqrt(jnp.float32(C)))
        iota = jax.lax.broadcasted_iota(jnp.int32, (KB, PROJ_DIM), 1)
        s = jnp.zeros((KB, PROJ_DIM), jnp.float32)
        for k in range(C):
            s = s + jnp.where(iota == idx[:, k:k + 1], sgn[:, k:k + 1], 0.0)
        s_ref[i % 2] = s.astype(jnp.bfloat16)

    @pl.when(i > 0)
    def _mm():
        xb = x_ref[...].astype(jnp.bfloat16)
        acc_ref[...] += jnp.dot(xb, s_ref[(i - 1) % 2],
                                preferred_element_type=jnp.float32)

    @pl.when(i == N_STEPS)
    def _done():
        o_ref[...] = acc_ref[...]


@jax.jit
def kernel(x, rand_indices, rand_signs):
    prev = lambda i: jnp.maximum(i - 1, 0)
    cur = lambda i: jnp.minimum(i, N_STEPS - 1)
    return pl.pallas_call(
        _body,
        grid=(N_STEPS + 1,),
        in_specs=[
            pl.BlockSpec((BATCH, KB), lambda i: (0, prev(i))),
            pl.BlockSpec((KB, C), lambda i: (cur(i), 0)),
            pl.BlockSpec((KB, C), lambda i: (cur(i), 0)),
        ],
        out_specs=pl.BlockSpec((BATCH, PROJ_DIM), lambda i: (0, 0)),
        out_shape=jax.ShapeDtypeStruct((BATCH, PROJ_DIM), jnp.float32),
        scratch_shapes=[
            pltpu.VMEM((BATCH, PROJ_DIM), jnp.float32),
            pltpu.VMEM((2, KB, PROJ_DIM), jnp.bfloat16),
        ],
    )(x, rand_indices, rand_signs)


# speedup vs baseline: 2.1534x; 1.1241x over previous
import jax
import jax.numpy as jnp
from jax.experimental import pallas as pl
from jax.experimental.pallas import tpu as pltpu

BATCH = 1024
ORIG_DIM = 16384
PROJ_DIM = 1024
C = 4
KB = 2048
N_STEPS = ORIG_DIM // KB


def _body(x_ref, idx_ref, sgn_ref, o_ref, acc_ref, s_ref):
    i = pl.program_id(0)

    @pl.when(i == 0)
    def _init():
        acc_ref[...] = jnp.zeros_like(acc_ref)
        s_ref[1] = jnp.zeros_like(s_ref[1])

    idx = idx_ref[...]
    sgn = sgn_ref[...] * (1.0 / jnp.sqrt(jnp.float32(C)))
    iota = jax.lax.broadcasted_iota(jnp.int32, (KB, PROJ_DIM), 1)
    s = jnp.where(iota == idx[:, 0:1], sgn[:, 0:1], 0.0)
    for k in range(1, C):
        s = s + jnp.where(iota == idx[:, k:k + 1], sgn[:, k:k + 1], 0.0)
    s_ref[i % 2] = s.astype(jnp.bfloat16)

    xb = x_ref[...].astype(jnp.bfloat16)
    acc_ref[...] += jnp.dot(xb, s_ref[(i - 1) % 2],
                            preferred_element_type=jnp.float32)

    @pl.when(i == N_STEPS)
    def _done():
        o_ref[...] = acc_ref[...]


@jax.jit
def kernel(x, rand_indices, rand_signs):
    prev = lambda i: jnp.maximum(i - 1, 0)
    cur = lambda i: jnp.minimum(i, N_STEPS - 1)
    return pl.pallas_call(
        _body,
        grid=(N_STEPS + 1,),
        in_specs=[
            pl.BlockSpec((BATCH, KB), lambda i: (0, prev(i))),
            pl.BlockSpec((KB, C), lambda i: (cur(i), 0)),
            pl.BlockSpec((KB, C), lambda i: (cur(i), 0)),
        ],
        out_specs=pl.BlockSpec((BATCH, PROJ_DIM), lambda i: (0, 0)),
        out_shape=jax.ShapeDtypeStruct((BATCH, PROJ_DIM), jnp.float32),
        scratch_shapes=[
            pltpu.VMEM((BATCH, PROJ_DIM), jnp.float32),
            pltpu.VMEM((2, KB, PROJ_DIM), jnp.bfloat16),
        ],
    )(x, rand_indices, rand_signs)
